# baseline (speedup 1.0000x reference)
import functools

import jax
import jax.numpy as jnp
from jax import lax
from jax.experimental import pallas as pl
from jax.experimental.pallas import tpu as pltpu
from jax.experimental.pallas import tpu_sc as plsc

NC, NS, L = 2, 16, 16
NW = NC * NS
CH = 128
ECH = 2048
PKB = 14

_SC_MESH = plsc.VectorSubcoreMesh(
    core_axis_name="c", subcore_axis_name="s", num_cores=NC, num_subcores=NS)
_SC_PARAMS = pltpu.CompilerParams(
    needs_layout_passes=False, use_tc_tiling_on_sc=False)


def _make_deg_kernel(n_pad, k_ch):
    chk = n_pad // NS

    def body(dst_hbm, ew_hbm, out_hbm, dstb, ewb, dloc, sumb, ldb, accs):
        c = lax.axis_index("c")
        s = lax.axis_index("s")

        def zero(i, _):
            dloc[pl.ds(i * L, L)] = jnp.zeros((L,), jnp.float32)
            return 0
        lax.fori_loop(0, n_pad // L, zero, 0)

        pltpu.sync_copy(dst_hbm.at[c, s], dstb)
        pltpu.sync_copy(ew_hbm.at[c, s], ewb)

        def chunk(k, _):
            for g in range(CH // L):
                idx = dstb[k, pl.ds(g * L, L)]
                w = ewb[k, pl.ds(g * L, L)]
                plsc.addupdate_scatter(dloc, [idx], w)
            return 0
        lax.fori_loop(0, k_ch, chunk, 0)

        pltpu.sync_copy(dloc, accs.at[s])
        plsc.subcore_barrier()

        def zero2(i, _):
            sumb[pl.ds(i * L, L)] = jnp.zeros((L,), jnp.float32)
            return 0
        lax.fori_loop(0, chk // L, zero2, 0)
        for t in range(NS):
            pltpu.sync_copy(accs.at[t, pl.ds(s * chk, chk)], ldb)

            def accum(i, _):
                sl = pl.ds(i * L, L)
                sumb[sl] = sumb[sl] + ldb[sl]
                return 0
            lax.fori_loop(0, chk // L, accum, 0)
        pltpu.sync_copy(sumb, out_hbm.at[c, pl.ds(s * chk, chk)])

    return pl.kernel(
        body,
        out_type=jax.ShapeDtypeStruct((NC, n_pad), jnp.float32),
        mesh=_SC_MESH,
        compiler_params=_SC_PARAMS,
        scratch_types=[
            pltpu.VMEM((k_ch, CH), jnp.int32),
            pltpu.VMEM((k_ch, CH), jnp.float32),
            pltpu.VMEM((n_pad,), jnp.float32),
            pltpu.VMEM((chk,), jnp.float32),
            pltpu.VMEM((chk,), jnp.float32),
            pltpu.VMEM_SHARED((NS, n_pad), jnp.float32),
        ],
    )


def _make_prop_kernel(np2, d, e_pad):
    f_pt = d // NW
    nch = e_pad // ECH
    assert e_pad % ECH == 0 and nch % 2 == 0
    ug = 8

    def body(y_hbm, pk_hbm, ew_hbm, out_hbm, yloc, accl, pkb, ewb,
             ps0, ps1, ws0, ws1):
        ps, ws = (ps0, ps1), (ws0, ws1)
        c = lax.axis_index("c")
        s = lax.axis_index("s")
        fbase = (s * NC + c) * (f_pt * np2)

        pltpu.sync_copy(y_hbm.at[pl.ds(fbase, f_pt * np2)], yloc)

        def zero(i, _):
            accl[pl.ds(i * L, L)] = jnp.zeros((L,), jnp.float32)
            return 0
        lax.fori_loop(0, f_pt * np2 // L, zero, 0)

        pltpu.async_copy(pk_hbm.at[pl.ds(0, ECH)], pkb.at[0], ps[0])
        pltpu.async_copy(ew_hbm.at[pl.ds(0, ECH)], ewb.at[0], ws[0])

        def pair(j2, _):
            for b in range(2):
                j = j2 * 2 + b
                pltpu.make_async_copy(pk_hbm.at[pl.ds(j * ECH, ECH)],
                                      pkb.at[b], ps[b]).wait()
                pltpu.make_async_copy(ew_hbm.at[pl.ds(j * ECH, ECH)],
                                      ewb.at[b], ws[b]).wait()
                bn = 1 - b

                @pl.when(j + 1 < nch)
                def _():
                    pltpu.async_copy(pk_hbm.at[pl.ds((j + 1) * ECH, ECH)],
                                     pkb.at[bn], ps[bn])
                    pltpu.async_copy(ew_hbm.at[pl.ds((j + 1) * ECH, ECH)],
                                     ewb.at[bn], ws[bn])

                def grp(gq, _):
                    for u in range(ug):
                        off = gq * (ug * L) + u * L
                        pk = pkb[b, pl.ds(off, L)]
                        wv = ewb[b, pl.ds(off, L)]
                        srcv = lax.shift_right_logical(pk, PKB)
                        dstv = lax.bitwise_and(pk, (1 << PKB) - 1)
                        for f in range(f_pt):
                            val = plsc.load_gather(yloc, [srcv + (f * np2)]) * wv
                            plsc.addupdate_scatter(accl, [dstv + (f * np2)], val)
                    return 0
                lax.fori_loop(0, ECH // (ug * L), grp, 0)
            return 0
        lax.fori_loop(0, nch // 2, pair, 0)

        pltpu.sync_copy(accl, out_hbm.at[pl.ds(fbase, f_pt * np2)])

    return pl.kernel(
        body,
        out_type=jax.ShapeDtypeStruct((d * np2,), jnp.float32),
        mesh=_SC_MESH,
        compiler_params=_SC_PARAMS,
        scratch_types=[
            pltpu.VMEM((f_pt * np2,), jnp.float32),
            pltpu.VMEM((f_pt * np2,), jnp.float32),
            pltpu.VMEM((2, ECH), jnp.int32),
            pltpu.VMEM((2, ECH), jnp.float32),
            *([pltpu.SemaphoreType.DMA] * 4),
        ],
    )


def _mm1_body(deg_ref, x_ref, w_ref, o_ref):
    deg = deg_ref[0:1, :] + deg_ref[1:2, :]
    d1 = lax.rsqrt(deg + 2.0)
    o_ref[...] = lax.dot_general(
        w_ref[...], x_ref[...], (((0,), (1,)), ((), ())),
        preferred_element_type=jnp.float32) * d1


def _comb_body(deg_ref, p_ref, y_ref, b_ref, w_ref, o_ref, *, fc, fn):
    deg = deg_ref[0:1, :] + deg_ref[1:2, :]
    dc = lax.rsqrt(deg + fc)
    dn = lax.rsqrt(deg + fn)
    h = jnp.maximum(dc * (p_ref[...] + fc * y_ref[...]) + b_ref[...], 0.0)
    o_ref[...] = lax.dot_general(
        w_ref[...], h, (((0,), (0,)), ((), ())),
        preferred_element_type=jnp.float32) * dn


def _final_body(deg_ref, p_ref, y_ref, b_ref, batch_ref, wl_ref, bl_ref,
                o_ref, seg_acc, cnt_acc, *, g):
    i = pl.program_id(0)

    @pl.when(i == 0)
    def _():
        seg_acc[...] = jnp.zeros_like(seg_acc)
        cnt_acc[...] = jnp.zeros_like(cnt_acc)

    deg = deg_ref[0:1, :] + deg_ref[1:2, :]
    dc = lax.rsqrt(deg + 1.0)
    h3 = dc * (p_ref[...] + y_ref[...]) + b_ref[...]
    oh = (batch_ref[...] == lax.broadcasted_iota(jnp.int32, (g, 1), 0)
          ).astype(jnp.float32)
    seg_acc[...] += lax.dot_general(h3, oh, (((1,), (1,)), ((), ())),
                                    preferred_element_type=jnp.float32)
    cnt_acc[...] += lax.dot_general(
        jnp.ones_like(batch_ref[...], jnp.float32), oh,
        (((1,), (1,)), ((), ())), preferred_element_type=jnp.float32)

    @pl.when(i == pl.num_programs(0) - 1)
    def _():
        pooled = seg_acc[...] / jnp.maximum(cnt_acc[...], 1.0)
        o_ref[...] = lax.dot_general(
            pooled, wl_ref[...], (((0,), (0,)), ((), ())),
            preferred_element_type=jnp.float32) + bl_ref[...]


def _col_spec(rows, bn):
    return pl.BlockSpec((rows, bn), lambda i: (0, i))


def _full_spec(shape):
    return pl.BlockSpec(shape, lambda i: tuple(0 for _ in shape))


def kernel(x, edge_index, batch, edge_weight, W1, b1, W2, b2, W3, b3, Wl, bl):
    n, d = x.shape
    h = W1.shape[1]
    g = 64
    c_out = Wl.shape[1]
    e = edge_weight.shape[0]

    np_ = -(-n // 2048) * 2048
    bn = np_ // 10
    grid = (np_ // bn,)

    k_ch = -(-e // (NW * CH))
    e_pad_d = NW * k_ch * CH
    dst = jnp.pad(edge_index[1], (0, e_pad_d - e)).reshape(NC, NS, k_ch, CH)
    ew_d = jnp.pad(edge_weight, (0, e_pad_d - e)).reshape(NC, NS, k_ch, CH)

    nch = -(-e // ECH)
    nch += nch % 2
    e_pad = nch * ECH
    pk = jnp.pad(
        jnp.left_shift(edge_index[0], PKB) | edge_index[1], (0, e_pad - e))
    ew_f = jnp.pad(edge_weight, (0, e_pad - e))

    x_p = jnp.pad(x, ((0, np_ - n), (0, 0)))
    batch_p = jnp.pad(batch, (0, np_ - n), constant_values=g).reshape(1, np_)

    degp = _make_deg_kernel(np_, k_ch)(dst, ew_d)

    prop = _make_prop_kernel(np_, d, e_pad)

    mm1 = pl.pallas_call(
        _mm1_body,
        grid=grid,
        in_specs=[_col_spec(NC, bn),
                  pl.BlockSpec((bn, d), lambda i: (i, 0)),
                  _full_spec((d, h))],
        out_specs=_col_spec(h, bn),
        out_shape=jax.ShapeDtypeStruct((h, np_), jnp.float32),
    )

    def comb(fc, fn, wnxt):
        return pl.pallas_call(
            functools.partial(_comb_body, fc=fc, fn=fn),
            grid=grid,
            in_specs=[
                _col_spec(NC, bn),
                _col_spec(h, bn),
                _col_spec(h, bn),
                _full_spec((h, 1)),
                _full_spec(wnxt.shape),
            ],
            out_specs=_col_spec(wnxt.shape[1], bn),
            out_shape=jax.ShapeDtypeStruct((wnxt.shape[1], np_), jnp.float32),
        )

    final = pl.pallas_call(
        functools.partial(_final_body, g=g),
        grid=grid,
        in_specs=[
            _col_spec(NC, bn),
            _col_spec(h, bn),
            _col_spec(h, bn),
            _full_spec((h, 1)),
            _col_spec(1, bn),
            _full_spec((h, c_out)),
            _full_spec((1, c_out)),
        ],
        out_specs=_full_spec((g, c_out)),
        out_shape=jax.ShapeDtypeStruct((g, c_out), jnp.float32),
        scratch_shapes=[
            pltpu.VMEM((h, g), jnp.float32),
            pltpu.VMEM((1, g), jnp.float32),
        ],
    )

    y1 = mm1(degp, x_p, W1)
    p1 = prop(y1.reshape(-1), pk, ew_f).reshape(h, np_)
    y2 = comb(2.0, 1.0, W2)(degp, p1, y1, b1.reshape(h, 1), W2)
    p2 = prop(y2.reshape(-1), pk, ew_f).reshape(h, np_)
    y3 = comb(1.0, 1.0, W3)(degp, p2, y2, b2.reshape(h, 1), W3)
    p3 = prop(y3.reshape(-1), pk, ew_f).reshape(h, np_)
    out = final(degp, p3, y3, b3.reshape(h, 1), batch_p,
                Wl, bl.reshape(1, c_out))
    return out


# speedup vs baseline: 2.1330x; 2.1330x over previous
import functools

import jax
import jax.numpy as jnp
from jax import lax
from jax.experimental import pallas as pl
from jax.experimental.pallas import tpu as pltpu
from jax.experimental.pallas import tpu_sc as plsc

NC, NS, L = 2, 16, 16
NW = NC * NS
CH = 128

_SC_MESH = plsc.VectorSubcoreMesh(
    core_axis_name="c", subcore_axis_name="s", num_cores=NC, num_subcores=NS)
_SC_PARAMS = pltpu.CompilerParams(
    needs_layout_passes=False, use_tc_tiling_on_sc=False)


def _make_deg_kernel(n, n_pad, k_ch):
    chk = n_pad // NS

    def body(dst_hbm, ew_hbm, out_hbm, dstb, ewb, dloc, sumb, ldb, accs):
        c = lax.axis_index("c")
        s = lax.axis_index("s")

        def zero(i, _):
            dloc[pl.ds(i * L, L)] = jnp.zeros((L,), jnp.float32)
            return 0
        lax.fori_loop(0, n_pad // L, zero, 0)

        pltpu.sync_copy(dst_hbm.at[c, s], dstb)
        pltpu.sync_copy(ew_hbm.at[c, s], ewb)

        def chunk(k, _):
            for g in range(CH // L):
                idx = dstb[k, pl.ds(g * L, L)]
                w = ewb[k, pl.ds(g * L, L)]
                plsc.addupdate_scatter(dloc, [idx], w)
            return 0
        lax.fori_loop(0, k_ch, chunk, 0)

        pltpu.sync_copy(dloc, accs.at[s])
        plsc.subcore_barrier()

        def zero2(i, _):
            sumb[pl.ds(i * L, L)] = jnp.zeros((L,), jnp.float32)
            return 0
        lax.fori_loop(0, chk // L, zero2, 0)
        for t in range(NS):
            pltpu.sync_copy(accs.at[t, pl.ds(s * chk, chk)], ldb)

            def accum(i, _):
                sl = pl.ds(i * L, L)
                sumb[sl] = sumb[sl] + ldb[sl]
                return 0
            lax.fori_loop(0, chk // L, accum, 0)
        pltpu.sync_copy(sumb, out_hbm.at[c, pl.ds(s * chk, chk)])

    return pl.kernel(
        body,
        out_type=jax.ShapeDtypeStruct((NC, n_pad), jnp.float32),
        mesh=_SC_MESH,
        compiler_params=_SC_PARAMS,
        scratch_types=[
            pltpu.VMEM((k_ch, CH), jnp.int32),
            pltpu.VMEM((k_ch, CH), jnp.float32),
            pltpu.VMEM((n_pad,), jnp.float32),
            pltpu.VMEM((chk,), jnp.float32),
            pltpu.VMEM((chk,), jnp.float32),
            pltpu.VMEM_SHARED((NS, n_pad), jnp.float32),
        ],
    )


CP = 32
NB = 3


def _make_prop_kernel(n, d, k3):
    rpt = n // NS
    assert rpt % CP == 0 and rpt % 8 == 0 and k3 % 6 == 0

    def body(y_hbm, src_hbm, dst_hbm, ew_hbm, out_hbm,
             srcb, dstb, ewb, rows, srows, *sems):
        gs, ss = sems[:2], sems[2:2 + NB]
        acc = sems[2 + NB]
        c = lax.axis_index("c")
        s = lax.axis_index("s")

        def zero(i, _):
            for f in range(d // L):
                srows[0, i, pl.ds(f * L, L)] = jnp.zeros((L,), jnp.float32)
            return 0
        lax.fori_loop(0, CP, zero, 0)
        for r in range(rpt // CP):
            pltpu.sync_copy(srows.at[0], acc.at[pl.ds(s * rpt + r * CP, CP)])

        pltpu.sync_copy(src_hbm.at[c, s], srcb)
        pltpu.sync_copy(dst_hbm.at[c, s], dstb)
        pltpu.sync_copy(ew_hbm.at[c, s], ewb)
        plsc.subcore_barrier()

        pltpu.async_copy(y_hbm.at[srcb.at[0]], rows.at[0], gs[0])

        def group(gi, _):
            for u in range(6):
                j = gi * 6 + u
                bg = u % 2
                bs = u % 3
                bgn = 1 - bg
                pltpu.make_async_copy(y_hbm.at[srcb.at[j]], rows.at[bg],
                                      gs[bg]).wait()

                @pl.when(j + 1 < k3)
                def _():
                    pltpu.async_copy(y_hbm.at[srcb.at[j + 1]], rows.at[bgn],
                                     gs[bgn])

                @pl.when(j >= NB)
                def _():
                    pltpu.make_async_copy(
                        srows.at[bs], acc.at[dstb.at[j - NB]], ss[bs]).wait()

                jvec = jnp.full((L,), j, jnp.int32)
                for i in range(CP):
                    w = plsc.load_gather(
                        ewb, [jvec, jnp.full((L,), i, jnp.int32)])
                    for f in range(d // (2 * L)):
                        pair = rows[bg, i, pl.ds(f * 2 * L, 2 * L)]
                        lo, hi = plsc.unpack(
                            pair, format=plsc.PackFormat.INTERLEAVED)
                        srows[bs, i, pl.ds(f * 2 * L, L)] = lo * w
                        srows[bs, i, pl.ds(f * 2 * L + L, L)] = hi * w
                pltpu.async_copy(srows.at[bs], acc.at[dstb.at[j]], ss[bs],
                                 add=True)
            return 0
        lax.fori_loop(0, k3 // 6, group, 0)
        for t in range(NB):
            pltpu.make_async_copy(srows.at[t], acc.at[dstb.at[k3 - NB + t]],
                                  ss[t]).wait()

        plsc.subcore_barrier()
        pltpu.sync_copy(acc.at[pl.ds(s * rpt, rpt)],
                        out_hbm.at[c, pl.ds(s * rpt, rpt)])

    return pl.kernel(
        body,
        out_type=jax.ShapeDtypeStruct((NC, n, d), jnp.float32),
        mesh=_SC_MESH,
        compiler_params=_SC_PARAMS,
        scratch_types=[
            pltpu.VMEM((k3, CP), jnp.int32),
            pltpu.VMEM((k3, CP), jnp.int32),
            pltpu.VMEM((k3, CP), jnp.float32),
            pltpu.VMEM((2, CP, d), jnp.bfloat16),
            pltpu.VMEM((NB, CP, d), jnp.float32),
            *([pltpu.SemaphoreType.DMA] * (2 + NB)),
            pltpu.VMEM_SHARED((n, d), jnp.float32),
        ],
    )


def _mm1_body(deg_ref, x_ref, w_ref, o_ref):
    deg = deg_ref[:, 0:1] + deg_ref[:, 1:2]
    d1 = lax.rsqrt(deg + 2.0)
    o_ref[...] = jnp.dot(x_ref[...], w_ref[...],
                         preferred_element_type=jnp.float32) * d1


def _comb_body(deg_ref, p_ref, b_ref, w_ref, o_ref, *, fc, fn):
    deg = deg_ref[:, 0:1] + deg_ref[:, 1:2]
    dc = lax.rsqrt(deg + fc)
    dn = lax.rsqrt(deg + fn)
    h = jnp.maximum(dc * (p_ref[0] + p_ref[1]) + b_ref[...], 0.0)
    o_ref[...] = jnp.dot(h, w_ref[...], preferred_element_type=jnp.float32) * dn


def _final_body(deg_ref, p_ref, b_ref, batch_ref, wl_ref, bl_ref,
                o_ref, seg_acc, cnt_acc, *, g):
    i = pl.program_id(0)

    @pl.when(i == 0)
    def _():
        seg_acc[...] = jnp.zeros_like(seg_acc)
        cnt_acc[...] = jnp.zeros_like(cnt_acc)

    deg = deg_ref[:, 0:1] + deg_ref[:, 1:2]
    dc = lax.rsqrt(deg + 1.0)
    h3 = dc * (p_ref[0] + p_ref[1]) + b_ref[...]
    oh = (batch_ref[...] == lax.broadcasted_iota(jnp.int32, (1, g), 1)
          ).astype(jnp.float32)
    seg_acc[...] += lax.dot_general(oh, h3, (((0,), (0,)), ((), ())),
                                    preferred_element_type=jnp.float32)
    cnt_acc[...] += lax.dot_general(oh, jnp.ones_like(batch_ref[...], jnp.float32),
                                    (((0,), (0,)), ((), ())),
                                    preferred_element_type=jnp.float32)

    @pl.when(i == pl.num_programs(0) - 1)
    def _():
        pooled = seg_acc[...] / jnp.maximum(cnt_acc[...], 1.0)
        o_ref[...] = jnp.dot(pooled, wl_ref[...],
                             preferred_element_type=jnp.float32) + bl_ref[...]


def _row_spec(bn, cols):
    return pl.BlockSpec((bn, cols), lambda i: (i, 0))


def _full_spec(shape):
    return pl.BlockSpec(shape, lambda i: tuple(0 for _ in shape))


def kernel(x, edge_index, batch, edge_weight, W1, b1, W2, b2, W3, b3, Wl, bl):
    n, d = x.shape
    h = W1.shape[1]
    g = 64
    c_out = Wl.shape[1]
    e = edge_weight.shape[0]

    np_ = -(-n // 2048) * 2048
    bn = np_ // 10
    grid = (np_ // bn,)

    k_ch = -(-e // (NW * CH))
    e_pad = NW * k_ch * CH

    dst = jnp.pad(edge_index[1], (0, e_pad - e)).reshape(NC, NS, k_ch, CH)
    ew = jnp.pad(edge_weight, (0, e_pad - e)).reshape(NC, NS, k_ch, CH)
    loop = jnp.arange(np_, dtype=edge_index.dtype)
    src_e = jnp.concatenate([edge_index[0], loop])
    dst_e = jnp.concatenate([edge_index[1], loop])
    e3 = e + np_
    k3 = -(-e3 // (NW * CP))
    k3 += (-k3) % 6
    e_pad3 = NW * k3 * CP
    src3 = jnp.pad(src_e, (0, e_pad3 - e3)).reshape(NC, NS, k3, CP)
    dst3 = jnp.pad(dst_e, (0, e_pad3 - e3)).reshape(NC, NS, k3, CP)

    def ew_with_fill(fill):
        ew_e = jnp.concatenate(
            [edge_weight, jnp.full((np_,), fill, jnp.float32)])
        return jnp.pad(ew_e, (0, e_pad3 - e3)).reshape(NC, NS, k3, CP)
    ew3a = ew_with_fill(2.0)
    ew3b = ew_with_fill(1.0)

    qq = jnp.arange(h)
    blk = (qq // 32) * 32
    r = qq % 32
    pinv = jnp.where(r % 2 == 0, blk + r // 2, blk + 16 + r // 2)
    W1p, W2p, W3p = W1[:, pinv], W2[:, pinv], W3[:, pinv]
    x_p = jnp.pad(x, ((0, np_ - n), (0, 0)))
    batch_p = jnp.pad(batch, (0, np_ - n), constant_values=g)

    degp = _make_deg_kernel(n, np_, k_ch)(dst, ew)
    deg_t = jnp.stack([degp[0], degp[1]], axis=1)

    prop = _make_prop_kernel(np_, d, k3)

    mm1 = pl.pallas_call(
        _mm1_body,
        grid=grid,
        in_specs=[_row_spec(bn, 2), _row_spec(bn, d), _full_spec((d, h))],
        out_specs=_row_spec(bn, h),
        out_shape=jax.ShapeDtypeStruct((np_, h), jnp.float32),
    )

    def comb(fc, fn, wnxt):
        return pl.pallas_call(
            functools.partial(_comb_body, fc=fc, fn=fn),
            grid=grid,
            in_specs=[
                _row_spec(bn, 2),
                pl.BlockSpec((NC, bn, h), lambda i: (0, i, 0)),
                _full_spec((1, h)),
                _full_spec(wnxt.shape),
            ],
            out_specs=_row_spec(bn, wnxt.shape[1]),
            out_shape=jax.ShapeDtypeStruct((np_, wnxt.shape[1]), jnp.float32),
        )

    final = pl.pallas_call(
        functools.partial(_final_body, g=g),
        grid=grid,
        in_specs=[
            _row_spec(bn, 2),
            pl.BlockSpec((NC, bn, h), lambda i: (0, i, 0)),
            _full_spec((1, h)),
            _row_spec(bn, 1),
            _full_spec((h, c_out)),
            _full_spec((1, c_out)),
        ],
        out_specs=_full_spec((g, c_out)),
        out_shape=jax.ShapeDtypeStruct((g, c_out), jnp.float32),
        scratch_shapes=[
            pltpu.VMEM((g, h), jnp.float32),
            pltpu.VMEM((g, 1), jnp.float32),
        ],
    )

    y1 = mm1(deg_t, x_p, W1p)
    p1 = prop(y1.astype(jnp.bfloat16), src3, dst3, ew3a)
    y2 = comb(2.0, 1.0, W2p)(deg_t, p1, b1.reshape(1, h), W2p)
    p2 = prop(y2.astype(jnp.bfloat16), src3, dst3, ew3b)
    y3 = comb(1.0, 1.0, W3p)(deg_t, p2, b2.reshape(1, h), W3p)
    p3 = prop(y3.astype(jnp.bfloat16), src3, dst3, ew3b)
    out = final(deg_t, p3, b3.reshape(1, h), batch_p.reshape(np_, 1),
                Wl, bl.reshape(1, c_out))
    return out
